# baseline (speedup 1.0000x reference)
import functools

import jax
import jax.numpy as jnp
from jax import lax
from jax.experimental import pallas as pl
from jax.experimental.pallas import tpu as pltpu
from jax.experimental.pallas import tpu_sc as plsc

B = 1024
HIST = 50
NEG = 10
D = 64
NODE_DIM = 1000000

NC = 2
NS = 16
NW = NC * NS

H_TOT = B * HIST
N_TOT = B * NEG
IDX_TOT = H_TOT + N_TOT + B + B
S_OFF = H_TOT + N_TOT
IDX_PAD = 65536

PACKED = NODE_DIM // 2
DPAD_ROWS = 7816

PW = IDX_PAD // NW
CH = 128
NCH = PW // CH
DCH = B // NW


def _gather_body(emb_hbm, dtab_hbm, idx_hbm, didx_hbm, rows_out, drows_out,
                 idx0, idx1, buf0, buf1, dbuf,
                 sem_g0, sem_g1, sem_o):
    wid = lax.axis_index("s") * NC + lax.axis_index("c")
    base = wid * PW
    idxs = (idx0, idx1)
    bufs = (buf0, buf1)
    sems = (sem_g0, sem_g1)

    gathers = [None] * NCH
    stores = [None] * NCH
    for c in range(NCH + 1):
        if c < NCH:
            pltpu.sync_copy(idx_hbm.at[pl.ds(base + c * CH, CH)], idxs[c % 2])
            if c >= 2:
                stores[c - 2].wait()
            gathers[c] = pltpu.async_copy(
                emb_hbm.at[idxs[c % 2]], bufs[c % 2], sems[c % 2])
        if c >= 1:
            gathers[c - 1].wait()
            stores[c - 1] = pltpu.async_copy(
                bufs[(c - 1) % 2],
                rows_out.at[pl.ds(base + (c - 1) * CH, CH)], sem_o)
    stores[NCH - 2].wait()
    stores[NCH - 1].wait()

    doff = wid * DCH
    pltpu.sync_copy(didx_hbm.at[pl.ds(doff, DCH)], idx0.at[pl.ds(0, DCH)])
    pltpu.async_copy(
        dtab_hbm.at[idx0.at[pl.ds(0, DCH)]], dbuf, sem_g0).wait()
    pltpu.sync_copy(dbuf, drows_out.at[pl.ds(doff, DCH)])


def _sc_gather(emb2, dtab_pad, idx2, didx):
    mesh = plsc.VectorSubcoreMesh(core_axis_name="c", subcore_axis_name="s")
    f = functools.partial(
        pl.kernel,
        mesh=mesh,
        out_type=[
            jax.ShapeDtypeStruct((IDX_PAD, 2 * D), jnp.float32),
            jax.ShapeDtypeStruct((B, 2 * D), jnp.float32),
        ],
        scratch_types=[
            pltpu.VMEM((CH,), jnp.int32),
            pltpu.VMEM((CH,), jnp.int32),
            pltpu.VMEM((CH, 2 * D), jnp.float32),
            pltpu.VMEM((CH, 2 * D), jnp.float32),
            pltpu.VMEM((DCH, 2 * D), jnp.float32),
            pltpu.SemaphoreType.DMA,
            pltpu.SemaphoreType.DMA,
            pltpu.SemaphoreType.DMA,
        ],
        compiler_params=pltpu.CompilerParams(use_tc_tiling_on_sc=True),
    )(_gather_body)
    return f(emb2, dtab_pad, idx2, didx)


BB = 128
GRID = B // BB


def _pick(rows2, par):
    lo = rows2[:, :D]
    hi = rows2[:, D:]
    return jnp.where(par > 0.5, hi, lo)


def _tc_body(h_ref, hp_ref, n_ref, np_ref, s_ref, sp_ref, t_ref, tp_ref,
             dr_ref, loh_ref, tt_ref, ht_ref, hm_ref, out_ref):
    s = _pick(s_ref[...], sp_ref[...])
    t = _pick(t_ref[...], tp_ref[...])
    h = _pick(h_ref[...], hp_ref[...]).reshape(BB, HIST, D)
    nn = _pick(n_ref[...], np_ref[...]).reshape(BB, NEG, D)
    delta = jnp.sum(dr_ref[...] * loh_ref[...], axis=1, keepdims=True)
    tt = tt_ref[...]
    ht = ht_ref[...]
    hm = hm_ref[...]

    d2_sh = jnp.sum((s[:, None, :] - h) ** 2, axis=2)
    att = jax.nn.softmax(-d2_sh, axis=1)
    c = att * jnp.exp(delta * jnp.abs(tt - ht)) * hm
    c0 = jnp.sum(c, axis=1, keepdims=True)
    h2 = jnp.sum(h * h, axis=2)
    c1 = jnp.sum(c * h2, axis=1, keepdims=True)
    hbar = jnp.sum(c[:, :, None] * h, axis=1)

    p_mu = -jnp.sum((s - t) ** 2, axis=1, keepdims=True)
    t2 = jnp.sum(t * t, axis=1, keepdims=True)
    ht_dot = jnp.sum(hbar * t, axis=1, keepdims=True)
    p_lam = p_mu - c1 + 2.0 * ht_dot - c0 * t2

    n_mu = -jnp.sum((s[:, None, :] - nn) ** 2, axis=2)
    n2 = jnp.sum(nn * nn, axis=2)
    hn_dot = jnp.sum(hbar[:, None, :] * nn, axis=2)
    n_lam = n_mu - c1 + 2.0 * hn_dot - c0 * n2

    pos = -jnp.log(jax.nn.sigmoid(p_lam) + 1e-6)
    neg = jnp.sum(jnp.log(jax.nn.sigmoid(-n_lam) + 1e-6),
                  axis=1, keepdims=True)
    out_ref[...] = pos - neg


def _tc_compute(rows2, par, drows, lane_oh, t_times, h_times, h_mask):
    return pl.pallas_call(
        _tc_body,
        grid=(GRID,),
        in_specs=[
            pl.BlockSpec((BB * HIST, 2 * D), lambda i: (i, 0)),
            pl.BlockSpec((BB * HIST, 1), lambda i: (i, 0)),
            pl.BlockSpec((BB * NEG, 2 * D),
                         lambda i: (H_TOT // (BB * NEG) + i, 0)),
            pl.BlockSpec((BB * NEG, 1),
                         lambda i: (H_TOT // (BB * NEG) + i, 0)),
            pl.BlockSpec((BB, 2 * D), lambda i: (S_OFF // BB + i, 0)),
            pl.BlockSpec((BB, 1), lambda i: (S_OFF // BB + i, 0)),
            pl.BlockSpec((BB, 2 * D), lambda i: ((S_OFF + B) // BB + i, 0)),
            pl.BlockSpec((BB, 1), lambda i: ((S_OFF + B) // BB + i, 0)),
            pl.BlockSpec((BB, 2 * D), lambda i: (i, 0)),
            pl.BlockSpec((BB, 2 * D), lambda i: (i, 0)),
            pl.BlockSpec((BB, 1), lambda i: (i, 0)),
            pl.BlockSpec((BB, HIST), lambda i: (i, 0)),
            pl.BlockSpec((BB, HIST), lambda i: (i, 0)),
        ],
        out_specs=pl.BlockSpec((BB, 1), lambda i: (i, 0)),
        out_shape=jax.ShapeDtypeStruct((B, 1), jnp.float32),
    )(rows2, par, rows2, par, rows2, par, rows2, par,
      drows, lane_oh, t_times, h_times, h_mask)


def kernel(s_nodes, t_nodes, t_times, h_nodes, h_times, h_time_mask,
           n_nodes, node_emb, delta_tab):
    s_idx = s_nodes.reshape(B)
    idx_all = jnp.concatenate([
        h_nodes.reshape(H_TOT), n_nodes.reshape(N_TOT),
        s_idx, t_nodes.reshape(B),
        jnp.zeros((IDX_PAD - IDX_TOT,), jnp.int32)])
    idx2 = idx_all // 2
    par = (idx_all % 2).astype(jnp.float32).reshape(IDX_PAD, 1)
    emb2 = node_emb.reshape(PACKED, 2 * D)
    dtab_pad = jnp.pad(delta_tab.reshape(NODE_DIM),
                       (0, DPAD_ROWS * 2 * D - NODE_DIM)).reshape(
                           DPAD_ROWS, 2 * D)
    didx = s_idx // (2 * D)
    lane_oh = (jnp.arange(2 * D, dtype=jnp.int32)[None, :]
               == (s_idx % (2 * D))[:, None]).astype(jnp.float32)
    rows2, drows = _sc_gather(emb2, dtab_pad, idx2, didx)
    out = _tc_compute(rows2, par, drows, lane_oh,
                      t_times, h_times, h_time_mask)
    return out.reshape(B)


# speedup vs baseline: 1.5160x; 1.5160x over previous
import functools

import jax
import jax.numpy as jnp
from jax import lax
from jax.experimental import pallas as pl
from jax.experimental.pallas import tpu as pltpu
from jax.experimental.pallas import tpu_sc as plsc

B = 1024
HIST = 50
NEG = 10
D = 64
NODE_DIM = 1000000

NC = 2
NS = 16
NW = NC * NS

H_TOT = B * HIST
N_TOT = B * NEG
IDX_TOT = H_TOT + N_TOT + B + B
S_OFF = H_TOT + N_TOT
IDX_PAD = 65536

DPAD_ROWS = 7816

PW = IDX_PAD // NW
CH = 256
NCH = PW // CH
GL = 16
DCH = B // NW


def _gather_body(emb_hbm, dtab_hbm, idx_hbm, didx_hbm, rows_out, drows_out,
                 idx_v, rows0, rows1, dbuf,
                 sem_g0, sem_g1, sem_o):
    wid = lax.axis_index("s") * NC + lax.axis_index("c")
    base = wid * PW
    bufs = (rows0, rows1)
    sems = (sem_g0, sem_g1)

    pltpu.sync_copy(idx_hbm.at[pl.ds(base, PW)], idx_v)

    for c in range(NCH + 1):
        if c < NCH:
            buf = bufs[c % 2]
            if c >= 2:
                pltpu.make_async_copy(
                    rows_out.at[pl.ds(0, CH)], buf, sem_o).wait()

            def issue(g, _, c=c, buf=buf, sem=sems[c % 2]):
                vec = idx_v[pl.ds(c * CH + g * GL, GL)]
                for k in range(GL):
                    i = vec[k]
                    pltpu.make_async_copy(
                        emb_hbm.at[pl.ds(i, 1), :],
                        buf.at[pl.ds(g * GL + k, 1), :], sem).start()
                return 0

            lax.fori_loop(0, CH // GL, issue, 0)
        if c >= 1:
            pbuf = bufs[(c - 1) % 2]
            pltpu.make_async_copy(
                rows_out.at[pl.ds(0, CH)], pbuf, sems[(c - 1) % 2]).wait()
            pltpu.make_async_copy(
                pbuf, rows_out.at[pl.ds(base + (c - 1) * CH, CH)],
                sem_o).start()

    pltpu.make_async_copy(rows_out.at[pl.ds(0, CH)], rows0, sem_o).wait()
    pltpu.make_async_copy(rows_out.at[pl.ds(0, CH)], rows1, sem_o).wait()

    doff = wid * DCH
    pltpu.sync_copy(didx_hbm.at[pl.ds(doff, DCH)], idx_v.at[pl.ds(0, DCH)])

    def issue_d(g, _):
        vec = idx_v[pl.ds(g * GL, GL)]
        for k in range(GL):
            i = vec[k]
            pltpu.make_async_copy(
                dtab_hbm.at[pl.ds(i, 1), :],
                dbuf.at[pl.ds(g * GL + k, 1), :], sem_g0).start()
        return 0

    lax.fori_loop(0, DCH // GL, issue_d, 0)
    pltpu.make_async_copy(drows_out.at[pl.ds(0, DCH)], dbuf, sem_g0).wait()
    pltpu.sync_copy(dbuf, drows_out.at[pl.ds(doff, DCH)])


def _sc_gather(node_emb, dtab_pad, idx_all, didx):
    mesh = plsc.VectorSubcoreMesh(core_axis_name="c", subcore_axis_name="s")
    f = functools.partial(
        pl.kernel,
        mesh=mesh,
        out_type=[
            jax.ShapeDtypeStruct((IDX_PAD, D), jnp.float32),
            jax.ShapeDtypeStruct((B, 2 * D), jnp.float32),
        ],
        scratch_types=[
            pltpu.VMEM((PW,), jnp.int32),
            pltpu.VMEM((CH, D), jnp.float32),
            pltpu.VMEM((CH, D), jnp.float32),
            pltpu.VMEM((DCH, 2 * D), jnp.float32),
            pltpu.SemaphoreType.DMA,
            pltpu.SemaphoreType.DMA,
            pltpu.SemaphoreType.DMA,
        ],
        compiler_params=pltpu.CompilerParams(use_tc_tiling_on_sc=True),
    )(_gather_body)
    return f(node_emb, dtab_pad, idx_all, didx)


BB = 128
GRID = B // BB


def _tc_body(h_ref, n_ref, s_ref, t_ref, dr_ref, li_ref, tt_ref, ht_ref,
             hm_ref, out_ref):
    s = s_ref[...]
    t = t_ref[...]
    h = h_ref[...].reshape(BB, HIST, D)
    nn = n_ref[...].reshape(BB, NEG, D)
    lanes = lax.broadcasted_iota(jnp.int32, (BB, 2 * D), 1)
    sel = (lanes == li_ref[...]).astype(jnp.float32)
    delta = jnp.sum(dr_ref[...] * sel, axis=1, keepdims=True)
    tt = tt_ref[...]
    ht = ht_ref[...]
    hm = hm_ref[...]

    d2_sh = jnp.sum((s[:, None, :] - h) ** 2, axis=2)
    att = jax.nn.softmax(-d2_sh, axis=1)
    c = att * jnp.exp(delta * jnp.abs(tt - ht)) * hm
    c0 = jnp.sum(c, axis=1, keepdims=True)
    h2 = jnp.sum(h * h, axis=2)
    c1 = jnp.sum(c * h2, axis=1, keepdims=True)
    hbar = jnp.sum(c[:, :, None] * h, axis=1)

    p_mu = -jnp.sum((s - t) ** 2, axis=1, keepdims=True)
    t2 = jnp.sum(t * t, axis=1, keepdims=True)
    ht_dot = jnp.sum(hbar * t, axis=1, keepdims=True)
    p_lam = p_mu - c1 + 2.0 * ht_dot - c0 * t2

    n_mu = -jnp.sum((s[:, None, :] - nn) ** 2, axis=2)
    n2 = jnp.sum(nn * nn, axis=2)
    hn_dot = jnp.sum(hbar[:, None, :] * nn, axis=2)
    n_lam = n_mu - c1 + 2.0 * hn_dot - c0 * n2

    pos = -jnp.log(jax.nn.sigmoid(p_lam) + 1e-6)
    neg = jnp.sum(jnp.log(jax.nn.sigmoid(-n_lam) + 1e-6),
                  axis=1, keepdims=True)
    out_ref[...] = pos - neg


def _tc_compute(rows, drows, li, t_times, h_times, h_mask):
    return pl.pallas_call(
        _tc_body,
        grid=(GRID,),
        in_specs=[
            pl.BlockSpec((BB * HIST, D), lambda i: (i, 0)),
            pl.BlockSpec((BB * NEG, D), lambda i: (H_TOT // (BB * NEG) + i, 0)),
            pl.BlockSpec((BB, D), lambda i: (S_OFF // BB + i, 0)),
            pl.BlockSpec((BB, D), lambda i: ((S_OFF + B) // BB + i, 0)),
            pl.BlockSpec((BB, 2 * D), lambda i: (i, 0)),
            pl.BlockSpec((BB, 1), lambda i: (i, 0)),
            pl.BlockSpec((BB, 1), lambda i: (i, 0)),
            pl.BlockSpec((BB, HIST), lambda i: (i, 0)),
            pl.BlockSpec((BB, HIST), lambda i: (i, 0)),
        ],
        out_specs=pl.BlockSpec((BB, 1), lambda i: (i, 0)),
        out_shape=jax.ShapeDtypeStruct((B, 1), jnp.float32),
    )(rows, rows, rows, rows, drows, li, t_times, h_times, h_mask)


def kernel(s_nodes, t_nodes, t_times, h_nodes, h_times, h_time_mask,
           n_nodes, node_emb, delta_tab):
    s_idx = s_nodes.reshape(B)
    idx_all = jnp.concatenate([
        h_nodes.reshape(H_TOT), n_nodes.reshape(N_TOT),
        s_idx, t_nodes.reshape(B),
        jnp.zeros((IDX_PAD - IDX_TOT,), jnp.int32)])
    dtab_pad = jnp.pad(delta_tab.reshape(NODE_DIM),
                       (0, DPAD_ROWS * 2 * D - NODE_DIM)).reshape(
                           DPAD_ROWS, 2 * D)
    didx = s_idx // (2 * D)
    li = (s_idx % (2 * D)).reshape(B, 1)
    rows, drows = _sc_gather(node_emb, dtab_pad, idx_all, didx)
    out = _tc_compute(rows, drows, li, t_times, h_times, h_time_mask)
    return out.reshape(B)


# speedup vs baseline: 2.0379x; 1.3442x over previous
import functools

import jax
import jax.numpy as jnp
from jax import lax
from jax.experimental import pallas as pl
from jax.experimental.pallas import tpu as pltpu
from jax.experimental.pallas import tpu_sc as plsc

B = 1024
HIST = 50
NEG = 10
D = 64
NODE_DIM = 1000000
HALF = NODE_DIM // 2

NC = 2
NS = 16
NW = NC * NS

H_TOT = B * HIST
N_TOT = B * NEG
IDX_TOT = H_TOT + N_TOT + B + B
S_OFF = H_TOT + N_TOT
IDX_PAD = 65536

DPAD_ROWS = 7816

PW = IDX_PAD // NW
CH = 256
NCH = PW // CH
GL = 16
DCH = B // NW


HALF = NODE_DIM // 2


def _gather_body(emb_hbm, dtab_hbm, idx_hbm, didx_hbm, rows_out, drows_out,
                 idx_v, rows0, rows1, dbuf,
                 sem_g0, sem_g1, sem_o):
    wid = lax.axis_index("s") * NC + lax.axis_index("c")
    base = wid * PW
    bufs = (rows0, rows1)
    sems = (sem_g0, sem_g1)

    pltpu.sync_copy(idx_hbm.at[pl.ds(base, PW)], idx_v)

    for c in range(NCH + 1):
        if c < NCH:
            buf = bufs[c % 2]
            if c >= 2:
                pltpu.make_async_copy(
                    rows_out.at[pl.ds(0, CH)], buf, sem_o).wait()

            def issue(g, _, c=c, buf=buf, sem=sems[c % 2]):
                vec = idx_v[pl.ds(c * CH + g * GL, GL)]
                for k in range(GL):
                    i = vec[k]
                    q = (i >= HALF).astype(jnp.int32)
                    r = i - q * HALF
                    pltpu.make_async_copy(
                        emb_hbm.at[q, pl.ds(r, 1), :],
                        buf.at[pl.ds(g * GL + k, 1), :], sem).start()
                return 0

            lax.fori_loop(0, CH // GL, issue, 0)
        if c >= 1:
            pbuf = bufs[(c - 1) % 2]
            pltpu.make_async_copy(
                rows_out.at[pl.ds(0, CH)], pbuf, sems[(c - 1) % 2]).wait()
            pltpu.make_async_copy(
                pbuf, rows_out.at[pl.ds(base + (c - 1) * CH, CH)],
                sem_o).start()

    pltpu.make_async_copy(rows_out.at[pl.ds(0, CH)], rows0, sem_o).wait()
    pltpu.make_async_copy(rows_out.at[pl.ds(0, CH)], rows1, sem_o).wait()

    doff = wid * DCH
    pltpu.sync_copy(didx_hbm.at[pl.ds(doff, DCH)], idx_v.at[pl.ds(0, DCH)])

    def issue_d(g, _):
        vec = idx_v[pl.ds(g * GL, GL)]
        for k in range(GL):
            i = vec[k]
            pltpu.make_async_copy(
                dtab_hbm.at[pl.ds(i, 1), :],
                dbuf.at[pl.ds(g * GL + k, 1), :], sem_g0).start()
        return 0

    lax.fori_loop(0, DCH // GL, issue_d, 0)
    pltpu.make_async_copy(drows_out.at[pl.ds(0, DCH)], dbuf, sem_g0).wait()
    pltpu.sync_copy(dbuf, drows_out.at[pl.ds(doff, DCH)])


def _sc_gather(node_emb, dtab_pad, idx_all, didx):
    mesh = plsc.VectorSubcoreMesh(core_axis_name="c", subcore_axis_name="s")
    f = functools.partial(
        pl.kernel,
        mesh=mesh,
        out_type=[
            jax.ShapeDtypeStruct((IDX_PAD, D), jnp.float32),
            jax.ShapeDtypeStruct((B, 2 * D), jnp.float32),
        ],
        scratch_types=[
            pltpu.VMEM((PW,), jnp.int32),
            pltpu.VMEM((CH, D), jnp.float32),
            pltpu.VMEM((CH, D), jnp.float32),
            pltpu.VMEM((DCH, 2 * D), jnp.float32),
            pltpu.SemaphoreType.DMA,
            pltpu.SemaphoreType.DMA,
            pltpu.SemaphoreType.DMA,
        ],
        compiler_params=pltpu.CompilerParams(use_tc_tiling_on_sc=True),
    )(_gather_body)
    return f(node_emb, dtab_pad, idx_all, didx)


BB = 128
GRID = B // BB


def _tc_body(h_ref, n_ref, s_ref, t_ref, dr_ref, li_ref, tt_ref, ht_ref,
             hm_ref, out_ref):
    s = s_ref[...]
    t = t_ref[...]
    h = h_ref[...].reshape(BB, HIST, D)
    nn = n_ref[...].reshape(BB, NEG, D)
    lanes = lax.broadcasted_iota(jnp.int32, (BB, 2 * D), 1)
    sel = (lanes == li_ref[...]).astype(jnp.float32)
    delta = jnp.sum(dr_ref[...] * sel, axis=1, keepdims=True)
    tt = tt_ref[...]
    ht = ht_ref[...]
    hm = hm_ref[...]

    d2_sh = jnp.sum((s[:, None, :] - h) ** 2, axis=2)
    att = jax.nn.softmax(-d2_sh, axis=1)
    c = att * jnp.exp(delta * jnp.abs(tt - ht)) * hm
    c0 = jnp.sum(c, axis=1, keepdims=True)
    h2 = jnp.sum(h * h, axis=2)
    c1 = jnp.sum(c * h2, axis=1, keepdims=True)
    hbar = jnp.sum(c[:, :, None] * h, axis=1)

    p_mu = -jnp.sum((s - t) ** 2, axis=1, keepdims=True)
    t2 = jnp.sum(t * t, axis=1, keepdims=True)
    ht_dot = jnp.sum(hbar * t, axis=1, keepdims=True)
    p_lam = p_mu - c1 + 2.0 * ht_dot - c0 * t2

    n_mu = -jnp.sum((s[:, None, :] - nn) ** 2, axis=2)
    n2 = jnp.sum(nn * nn, axis=2)
    hn_dot = jnp.sum(hbar[:, None, :] * nn, axis=2)
    n_lam = n_mu - c1 + 2.0 * hn_dot - c0 * n2

    pos = -jnp.log(jax.nn.sigmoid(p_lam) + 1e-6)
    neg = jnp.sum(jnp.log(jax.nn.sigmoid(-n_lam) + 1e-6),
                  axis=1, keepdims=True)
    out_ref[...] = pos - neg


def _tc_compute(rows, drows, li, t_times, h_times, h_mask):
    return pl.pallas_call(
        _tc_body,
        grid=(GRID,),
        in_specs=[
            pl.BlockSpec((BB * HIST, D), lambda i: (i, 0)),
            pl.BlockSpec((BB * NEG, D), lambda i: (H_TOT // (BB * NEG) + i, 0)),
            pl.BlockSpec((BB, D), lambda i: (S_OFF // BB + i, 0)),
            pl.BlockSpec((BB, D), lambda i: ((S_OFF + B) // BB + i, 0)),
            pl.BlockSpec((BB, 2 * D), lambda i: (i, 0)),
            pl.BlockSpec((BB, 1), lambda i: (i, 0)),
            pl.BlockSpec((BB, 1), lambda i: (i, 0)),
            pl.BlockSpec((BB, HIST), lambda i: (i, 0)),
            pl.BlockSpec((BB, HIST), lambda i: (i, 0)),
        ],
        out_specs=pl.BlockSpec((BB, 1), lambda i: (i, 0)),
        out_shape=jax.ShapeDtypeStruct((B, 1), jnp.float32),
    )(rows, rows, rows, rows, drows, li, t_times, h_times, h_mask)


def kernel(s_nodes, t_nodes, t_times, h_nodes, h_times, h_time_mask,
           n_nodes, node_emb, delta_tab):
    s_idx = s_nodes.reshape(B)
    idx_all = jnp.concatenate([
        h_nodes.reshape(H_TOT), n_nodes.reshape(N_TOT),
        s_idx, t_nodes.reshape(B),
        jnp.zeros((IDX_PAD - IDX_TOT,), jnp.int32)])
    dtab_pad = jnp.pad(delta_tab.reshape(NODE_DIM),
                       (0, DPAD_ROWS * 2 * D - NODE_DIM)).reshape(
                           DPAD_ROWS, 2 * D)
    didx = s_idx // (2 * D)
    li = (s_idx % (2 * D)).reshape(B, 1)
    rows, drows = _sc_gather(node_emb.reshape(2, HALF, D), dtab_pad, idx_all, didx)
    out = _tc_compute(rows, drows, li, t_times, h_times, h_time_mask)
    return out.reshape(B)


# speedup vs baseline: 2.0614x; 1.0115x over previous
import functools

import jax
import jax.numpy as jnp
from jax import lax
from jax.experimental import pallas as pl
from jax.experimental.pallas import tpu as pltpu
from jax.experimental.pallas import tpu_sc as plsc

B = 1024
HIST = 50
NEG = 10
D = 64
NODE_DIM = 1000000
HALF = NODE_DIM // 2

NC = 2
NS = 16
NW = NC * NS

H_TOT = B * HIST
N_TOT = B * NEG
IDX_TOT = H_TOT + N_TOT + B + B
S_OFF = H_TOT + N_TOT
IDX_PAD = 65536

DPAD_ROWS = 7816

PW = IDX_PAD // NW
CH = 256
NCH = PW // CH
GL = 16
DCH = B // NW


HALF = NODE_DIM // 2


def _gather_body(emb_hbm, dtab_hbm, idx_hbm, didx_hbm, rows_out, drows_out,
                 idx_v, rows0, rows1, dbuf,
                 sem_g0, sem_g1, sem_o):
    wid = lax.axis_index("s") * NC + lax.axis_index("c")
    base = wid * PW
    bufs = (rows0, rows1)
    sems = (sem_g0, sem_g1)

    pltpu.sync_copy(idx_hbm.at[pl.ds(base, PW)], idx_v)

    for c in range(NCH + 1):
        if c < NCH:
            buf = bufs[c % 2]
            if c >= 2:
                pltpu.make_async_copy(
                    rows_out.at[pl.ds(0, CH)], buf, sem_o).wait()

            def issue(g, _, c=c, buf=buf, sem=sems[c % 2]):
                vec = idx_v[pl.ds(c * CH + g * GL, GL)]
                for k in range(GL):
                    i = vec[k]
                    q = (i >= HALF).astype(jnp.int32)
                    r = i - q * HALF
                    pltpu.make_async_copy(
                        emb_hbm.at[q, pl.ds(r, 1), :],
                        buf.at[pl.ds(g * GL + k, 1), :], sem).start()
                return 0

            lax.fori_loop(0, CH // GL, issue, 0)
        if c >= 1:
            pbuf = bufs[(c - 1) % 2]
            pltpu.make_async_copy(
                rows_out.at[pl.ds(0, CH)], pbuf, sems[(c - 1) % 2]).wait()
            pltpu.make_async_copy(
                pbuf, rows_out.at[pl.ds(base + (c - 1) * CH, CH)],
                sem_o).start()

    pltpu.make_async_copy(rows_out.at[pl.ds(0, CH)], rows0, sem_o).wait()
    pltpu.make_async_copy(rows_out.at[pl.ds(0, CH)], rows1, sem_o).wait()

    doff = wid * DCH
    pltpu.sync_copy(didx_hbm.at[pl.ds(doff, DCH)], idx_v.at[pl.ds(0, DCH)])

    def issue_d(g, _):
        vec = idx_v[pl.ds(g * GL, GL)]
        for k in range(GL):
            i = vec[k]
            pltpu.make_async_copy(
                dtab_hbm.at[pl.ds(i, 1), :],
                dbuf.at[pl.ds(g * GL + k, 1), :], sem_g0).start()
        return 0

    lax.fori_loop(0, DCH // GL, issue_d, 0)
    pltpu.make_async_copy(drows_out.at[pl.ds(0, DCH)], dbuf, sem_g0).wait()
    pltpu.sync_copy(dbuf, drows_out.at[pl.ds(doff, DCH)])


def _sc_gather(node_emb, dtab_pad, idx_all, didx):
    mesh = plsc.VectorSubcoreMesh(core_axis_name="c", subcore_axis_name="s")
    f = functools.partial(
        pl.kernel,
        mesh=mesh,
        out_type=[
            jax.ShapeDtypeStruct((IDX_PAD, D), jnp.float32),
            jax.ShapeDtypeStruct((B, 2 * D), jnp.float32),
        ],
        scratch_types=[
            pltpu.VMEM((PW,), jnp.int32),
            pltpu.VMEM((CH, D), jnp.float32),
            pltpu.VMEM((CH, D), jnp.float32),
            pltpu.VMEM((DCH, 2 * D), jnp.float32),
            pltpu.SemaphoreType.DMA,
            pltpu.SemaphoreType.DMA,
            pltpu.SemaphoreType.DMA,
        ],
        compiler_params=pltpu.CompilerParams(use_tc_tiling_on_sc=True),
    )(_gather_body)
    return f(node_emb, dtab_pad, idx_all, didx)


BB = 128
GRID = B // BB


def _tc_body(h_ref, n_ref, s_ref, t_ref, dr_ref, li_ref, tt_ref, ht_ref,
             hm_ref, out_ref):
    s = s_ref[...]
    t = t_ref[...]
    h = h_ref[...].reshape(BB, HIST, D)
    nn = n_ref[...].reshape(BB, NEG, D)
    lanes = lax.broadcasted_iota(jnp.int32, (BB, 2 * D), 1)
    sel = (lanes == li_ref[...]).astype(jnp.float32)
    delta = jnp.sum(dr_ref[...] * sel, axis=1, keepdims=True)
    tt = tt_ref[...]
    ht = ht_ref[...]
    hm = hm_ref[...]

    d2_sh = jnp.sum((s[:, None, :] - h) ** 2, axis=2)
    att = jax.nn.softmax(-d2_sh, axis=1)
    c = att * jnp.exp(delta * jnp.abs(tt - ht)) * hm
    c0 = jnp.sum(c, axis=1, keepdims=True)
    h2 = jnp.sum(h * h, axis=2)
    c1 = jnp.sum(c * h2, axis=1, keepdims=True)
    hbar = jnp.sum(c[:, :, None] * h, axis=1)

    p_mu = -jnp.sum((s - t) ** 2, axis=1, keepdims=True)
    t2 = jnp.sum(t * t, axis=1, keepdims=True)
    ht_dot = jnp.sum(hbar * t, axis=1, keepdims=True)
    p_lam = p_mu - c1 + 2.0 * ht_dot - c0 * t2

    n_mu = -jnp.sum((s[:, None, :] - nn) ** 2, axis=2)
    n2 = jnp.sum(nn * nn, axis=2)
    hn_dot = jnp.sum(hbar[:, None, :] * nn, axis=2)
    n_lam = n_mu - c1 + 2.0 * hn_dot - c0 * n2

    pos = -jnp.log(jax.nn.sigmoid(p_lam) + 1e-6)
    neg = jnp.sum(jnp.log(jax.nn.sigmoid(-n_lam) + 1e-6),
                  axis=1, keepdims=True)
    out_ref[...] = pos - neg


def _tc_compute(rows, drows, li, t_times, h_times, h_mask):
    return pl.pallas_call(
        _tc_body,
        grid=(GRID,),
        in_specs=[
            pl.BlockSpec((BB * HIST, D), lambda i: (i, 0)),
            pl.BlockSpec((BB * NEG, D), lambda i: (H_TOT // (BB * NEG) + i, 0)),
            pl.BlockSpec((BB, D), lambda i: (S_OFF // BB + i, 0)),
            pl.BlockSpec((BB, D), lambda i: ((S_OFF + B) // BB + i, 0)),
            pl.BlockSpec((BB, 2 * D), lambda i: (i, 0)),
            pl.BlockSpec((BB, 1), lambda i: (i, 0)),
            pl.BlockSpec((BB, 1), lambda i: (i, 0)),
            pl.BlockSpec((BB, HIST), lambda i: (i, 0)),
            pl.BlockSpec((BB, HIST), lambda i: (i, 0)),
        ],
        out_specs=pl.BlockSpec((BB, 1), lambda i: (i, 0)),
        out_shape=jax.ShapeDtypeStruct((B, 1), jnp.float32),
    )(rows, rows, rows, rows, drows, li, t_times, h_times, h_mask)


def kernel(s_nodes, t_nodes, t_times, h_nodes, h_times, h_time_mask,
           n_nodes, node_emb, delta_tab):
    s_idx = s_nodes.reshape(B)
    idx_all = jnp.concatenate([
        h_nodes.reshape(H_TOT), n_nodes.reshape(N_TOT),
        s_idx, t_nodes.reshape(B),
        jnp.zeros((IDX_PAD - IDX_TOT,), jnp.int32)])
    dtab_pad = jnp.pad(delta_tab,
                       ((0, DPAD_ROWS * 2 * D - NODE_DIM), (0, 0))).reshape(
                           DPAD_ROWS, 2 * D)
    didx = s_idx // (2 * D)
    li = (s_idx % (2 * D)).reshape(B, 1)
    rows, drows = _sc_gather(node_emb.reshape(2, HALF, D), dtab_pad, idx_all, didx)
    out = _tc_compute(rows, drows, li, t_times, h_times, h_time_mask)
    return out.reshape(B)
